# Initial kernel scaffold; baseline (speedup 1.0000x reference)
import jax
import jax.numpy as jnp
from jax.experimental import pallas as pl


def kernel(tables, gamma, beta, dense, ids, hist_ids):
    raise NotImplementedError("write your pallas kernel here")


# speedup vs baseline: 2.5312x; 2.5312x over previous
import functools

import jax
import jax.numpy as jnp
from jax.experimental import pallas as pl
from jax.experimental.pallas import tpu as pltpu
from jax.experimental.pallas import tpu_sc as plsc

_GATHER_WINDOW = 128
_BT = 256


def _sc_gather(tflat, idx_flat):
    n_idx = idx_flat.shape[1]
    d = tflat.shape[1]
    mesh = plsc.VectorSubcoreMesh(
        core_axis_name="core", subcore_axis_name="subcore"
    )

    @functools.partial(
        pl.kernel,
        out_type=jax.ShapeDtypeStruct((n_idx, d), tflat.dtype),
        mesh=mesh,
        compiler_params=pltpu.CompilerParams(use_tc_tiling_on_sc=False),
    )
    def _kernel(x_hbm, i_hbm, o_hbm):
        def body(i_vmem, o_vmem):
            pltpu.sync_copy(x_hbm.at[i_vmem.at[0]], o_vmem)

        pltpu.emit_pipeline(
            body,
            grid=(n_idx // _GATHER_WINDOW,),
            in_specs=[
                pl.BlockSpec((1, _GATHER_WINDOW), index_map=lambda i: (0, i))
            ],
            out_specs=[
                pl.BlockSpec((_GATHER_WINDOW, d), index_map=lambda i: (i, 0))
            ],
            core_axis_name=("core", "subcore"),
            dimension_semantics=(pltpu.PARALLEL,),
        )(i_hbm, o_hbm)

    return _kernel(tflat, idx_flat)


def _finish_body(x_ref, hb_ref, d_ref, t00_ref, a1_ref, a2_ref,
                 gp_ref, gf_ref, gd_ref, bp_ref, bf_ref, bd_ref, o_ref,
                 *, hist, n_fields, d_emb, ln_dim):
    nf_cols = n_fields * d_emb
    x = x_ref[...]
    xf = x[:, :nf_cols]
    sums = jnp.dot(x, a1_ref[...], preferred_element_type=jnp.float32)
    h1u = sums[:, :d_emb]
    sf = sums[:, d_emb:]
    qf = jnp.dot(xf * xf, a2_ref[...], preferred_element_type=jnp.float32)

    hb = hb_ref[...]
    cnt = jnp.sum((hb > 0).astype(jnp.float32), axis=1, keepdims=True)
    n0 = hist - cnt
    h1 = h1u - n0 * t00_ref[...]
    pooled = h1 / (cnt + 1e-8)

    s = pooled + sf
    q = pooled * pooled + qf
    dn = d_ref[...]
    sum_x = jnp.sum(s, axis=1, keepdims=True) + jnp.sum(
        dn, axis=1, keepdims=True)
    sumsq_x = jnp.sum(q, axis=1, keepdims=True) + jnp.sum(
        dn * dn, axis=1, keepdims=True)
    mu = sum_x / ln_dim
    var = sumsq_x / ln_dim - mu * mu
    rstd = jax.lax.rsqrt(var + 1e-5)
    cross = 0.5 * (jnp.sum(s * s, axis=1, keepdims=True)
                   - jnp.sum(q, axis=1, keepdims=True))

    ln_p = gp_ref[...] * ((pooled - mu) * rstd) + bp_ref[...]
    ln_f = gf_ref[...] * ((xf - mu) * rstd) + bf_ref[...]
    ln_d = gd_ref[...] * ((dn - mu) * rstd) + bd_ref[...]
    o_ref[...] = jnp.concatenate([ln_p, ln_f, ln_d, cross], axis=1)


def kernel(tables, gamma, beta, dense, ids, hist_ids):
    f_all, v, d = tables.shape
    b = dense.shape[0]
    n_fields = f_all - 1
    hist = hist_ids.shape[1]
    n_dense = dense.shape[1]
    row = (n_fields + hist) * d
    nf_cols = n_fields * d
    ln_dim = f_all * d + n_dense

    ids32 = ids.astype(jnp.int32)
    hist32 = hist_ids.astype(jnp.int32)
    offs = (jnp.arange(1, f_all, dtype=jnp.int32) * v)[None, :]
    idx_all = jnp.concatenate([ids32 + offs, hist32], axis=1)
    idx_flat = idx_all.reshape(1, b * (n_fields + hist))
    tflat = tables.reshape(f_all * v, d)

    gathered = _sc_gather(tflat, idx_flat)
    x = gathered.reshape(b, row)

    eye = jnp.eye(d, dtype=jnp.float32)
    zed = jnp.zeros((d, d), dtype=jnp.float32)
    a1 = jnp.concatenate(
        [
            jnp.tile(jnp.concatenate([zed, eye], axis=1), (n_fields, 1)),
            jnp.tile(jnp.concatenate([eye, zed], axis=1), (hist, 1)),
        ],
        axis=0,
    )
    a2 = jnp.tile(eye, (n_fields, 1))
    t00 = tables[0, 0].reshape(1, d)
    gp, gf, gd = (gamma[:d].reshape(1, d),
                  gamma[d:d + nf_cols].reshape(1, nf_cols),
                  gamma[d + nf_cols:].reshape(1, n_dense))
    bp, bf, bd = (beta[:d].reshape(1, d),
                  beta[d:d + nf_cols].reshape(1, nf_cols),
                  beta[d + nf_cols:].reshape(1, n_dense))

    body = functools.partial(_finish_body, hist=float(hist),
                             n_fields=n_fields, d_emb=d, ln_dim=float(ln_dim))
    const = lambda shape: pl.BlockSpec(shape, lambda i: (0, 0))
    out = pl.pallas_call(
        body,
        grid=(b // _BT,),
        in_specs=[
            pl.BlockSpec((_BT, row), lambda i: (i, 0)),
            pl.BlockSpec((_BT, hist), lambda i: (i, 0)),
            pl.BlockSpec((_BT, n_dense), lambda i: (i, 0)),
            const((1, d)),
            const((row, 2 * d)),
            const((nf_cols, d)),
            const((1, d)),
            const((1, nf_cols)),
            const((1, n_dense)),
            const((1, d)),
            const((1, nf_cols)),
            const((1, n_dense)),
        ],
        out_specs=pl.BlockSpec((_BT, ln_dim + 1), lambda i: (i, 0)),
        out_shape=jax.ShapeDtypeStruct((b, ln_dim + 1), jnp.float32),
    )(x, hist32, dense, t00, a1, a2, gp, gf, gd, bp, bf, bd)
    return out


# speedup vs baseline: 2.5746x; 1.0172x over previous
import functools

import jax
import jax.numpy as jnp
from jax.experimental import pallas as pl
from jax.experimental.pallas import tpu as pltpu
from jax.experimental.pallas import tpu_sc as plsc

_GATHER_WINDOW = 128
_BT = 256
_CHUNKS = 4


def _sc_gather(tflat, idx_flat):
    n_idx = idx_flat.shape[1]
    d = tflat.shape[1]
    mesh = plsc.VectorSubcoreMesh(
        core_axis_name="core", subcore_axis_name="subcore"
    )

    @functools.partial(
        pl.kernel,
        out_type=jax.ShapeDtypeStruct((n_idx, d), tflat.dtype),
        mesh=mesh,
        compiler_params=pltpu.CompilerParams(use_tc_tiling_on_sc=False),
    )
    def _kernel(x_hbm, i_hbm, o_hbm):
        def body(i_vmem, o_vmem):
            pltpu.sync_copy(x_hbm.at[i_vmem.at[0]], o_vmem)

        pltpu.emit_pipeline(
            body,
            grid=(n_idx // _GATHER_WINDOW,),
            in_specs=[
                pl.BlockSpec((1, _GATHER_WINDOW), index_map=lambda i: (0, i))
            ],
            out_specs=[
                pl.BlockSpec((_GATHER_WINDOW, d), index_map=lambda i: (i, 0))
            ],
            core_axis_name=("core", "subcore"),
            dimension_semantics=(pltpu.PARALLEL,),
        )(i_hbm, o_hbm)

    return _kernel(tflat, idx_flat)


def _finish_body(x_ref, hb_ref, d_ref, t00_ref, a1_ref, a2_ref,
                 gp_ref, gf_ref, gd_ref, bp_ref, bf_ref, bd_ref, o_ref,
                 *, hist, n_fields, d_emb, ln_dim):
    nf_cols = n_fields * d_emb
    x = x_ref[...]
    xf = x[:, :nf_cols]
    sums = jnp.dot(x, a1_ref[...], preferred_element_type=jnp.float32)
    h1u = sums[:, :d_emb]
    sf = sums[:, d_emb:]
    qf = jnp.dot(xf * xf, a2_ref[...], preferred_element_type=jnp.float32)

    hb = hb_ref[...]
    cnt = jnp.sum((hb > 0).astype(jnp.float32), axis=1, keepdims=True)
    n0 = hist - cnt
    h1 = h1u - n0 * t00_ref[...]
    pooled = h1 / (cnt + 1e-8)

    s = pooled + sf
    q = pooled * pooled + qf
    dn = d_ref[...]
    sum_x = jnp.sum(s, axis=1, keepdims=True) + jnp.sum(
        dn, axis=1, keepdims=True)
    sumsq_x = jnp.sum(q, axis=1, keepdims=True) + jnp.sum(
        dn * dn, axis=1, keepdims=True)
    mu = sum_x / ln_dim
    var = sumsq_x / ln_dim - mu * mu
    rstd = jax.lax.rsqrt(var + 1e-5)
    cross = 0.5 * (jnp.sum(s * s, axis=1, keepdims=True)
                   - jnp.sum(q, axis=1, keepdims=True))

    ln_p = gp_ref[...] * ((pooled - mu) * rstd) + bp_ref[...]
    ln_f = gf_ref[...] * ((xf - mu) * rstd) + bf_ref[...]
    ln_d = gd_ref[...] * ((dn - mu) * rstd) + bd_ref[...]
    o_ref[...] = jnp.concatenate([ln_p, ln_f, ln_d, cross], axis=1)


def kernel(tables, gamma, beta, dense, ids, hist_ids):
    f_all, v, d = tables.shape
    b = dense.shape[0]
    n_fields = f_all - 1
    hist = hist_ids.shape[1]
    n_dense = dense.shape[1]
    row = (n_fields + hist) * d
    nf_cols = n_fields * d
    ln_dim = f_all * d + n_dense

    ids32 = ids.astype(jnp.int32)
    hist32 = hist_ids.astype(jnp.int32)
    offs = (jnp.arange(1, f_all, dtype=jnp.int32) * v)[None, :]
    idx_all = jnp.concatenate([ids32 + offs, hist32], axis=1)
    tflat = tables.reshape(f_all * v, d)

    eye = jnp.eye(d, dtype=jnp.float32)
    zed = jnp.zeros((d, d), dtype=jnp.float32)
    a1 = jnp.concatenate(
        [
            jnp.tile(jnp.concatenate([zed, eye], axis=1), (n_fields, 1)),
            jnp.tile(jnp.concatenate([eye, zed], axis=1), (hist, 1)),
        ],
        axis=0,
    )
    a2 = jnp.tile(eye, (n_fields, 1))
    t00 = tables[0, 0].reshape(1, d)
    gp, gf, gd = (gamma[:d].reshape(1, d),
                  gamma[d:d + nf_cols].reshape(1, nf_cols),
                  gamma[d + nf_cols:].reshape(1, n_dense))
    bp, bf, bd = (beta[:d].reshape(1, d),
                  beta[d:d + nf_cols].reshape(1, nf_cols),
                  beta[d + nf_cols:].reshape(1, n_dense))

    body = functools.partial(_finish_body, hist=float(hist),
                             n_fields=n_fields, d_emb=d, ln_dim=float(ln_dim))
    const = lambda shape: pl.BlockSpec(shape, lambda i: (0, 0))

    bc = b // _CHUNKS
    outs = []
    for c in range(_CHUNKS):
        sl = slice(c * bc, (c + 1) * bc)
        idx_flat = idx_all[sl].reshape(1, bc * (n_fields + hist))
        gathered = _sc_gather(tflat, idx_flat)
        x = gathered.reshape(bc, row)
        out_c = pl.pallas_call(
            body,
            grid=(bc // _BT,),
            in_specs=[
                pl.BlockSpec((_BT, row), lambda i: (i, 0)),
                pl.BlockSpec((_BT, hist), lambda i: (i, 0)),
                pl.BlockSpec((_BT, n_dense), lambda i: (i, 0)),
                const((1, d)),
                const((row, 2 * d)),
                const((nf_cols, d)),
                const((1, d)),
                const((1, nf_cols)),
                const((1, n_dense)),
                const((1, d)),
                const((1, nf_cols)),
                const((1, n_dense)),
            ],
            out_specs=pl.BlockSpec((_BT, ln_dim + 1), lambda i: (i, 0)),
            out_shape=jax.ShapeDtypeStruct((bc, ln_dim + 1), jnp.float32),
        )(x, hist32[sl], dense[sl], t00, a1, a2, gp, gf, gd, bp, bf, bd)
        outs.append(out_c)
    return jnp.concatenate(outs, axis=0)


# speedup vs baseline: 3.0061x; 1.1676x over previous
import functools

import jax
import jax.numpy as jnp
from jax.experimental import pallas as pl
from jax.experimental.pallas import tpu as pltpu
from jax.experimental.pallas import tpu_sc as plsc

_GATHER_WINDOW = 128
_BT = 256
_CHUNKS = 4


_TCH = 512


def _transpose_tables(tables):
    f_all, v, d = tables.shape
    tt = jnp.transpose(tables, (0, 2, 1))
    pack = 128 // d
    rows_f = v // pack
    n_full = v // _TCH
    tail = v - n_full * _TCH

    def sel(w):
        wg = w // pack
        r = jnp.arange(w, dtype=jnp.int32)
        src_of_c = pack * (r % wg) + r // wg
        return (jnp.arange(w, dtype=jnp.int32)[:, None]
                == src_of_c[None, :]).astype(jnp.float32)

    s_full = sel(_TCH)
    s_tail = sel(tail)

    def body(t_ref, sf_ref, st_ref, o_ref):
        def emit(k, w, s):
            xc = t_ref[0, :, pl.ds(k * _TCH, w)]
            y = jnp.dot(xc, s, preferred_element_type=jnp.float32)
            wg = w // pack
            for q in range(pack):
                o_ref[pl.ds(k * (_TCH // pack), wg), q * d:(q + 1) * d] = (
                    y[:, q * wg:(q + 1) * wg].T)

        for k in range(n_full):
            emit(k, _TCH, sf_ref[...])
        if tail:
            emit(n_full, tail, st_ref[...])

    return pl.pallas_call(
        body,
        grid=(f_all,),
        in_specs=[
            pl.BlockSpec((1, d, v), lambda f: (f, 0, 0),
                         pipeline_mode=pl.Buffered(buffer_count=1)),
            pl.BlockSpec((_TCH, _TCH), lambda f: (0, 0),
                         pipeline_mode=pl.Buffered(buffer_count=1)),
            pl.BlockSpec((tail, tail), lambda f: (0, 0),
                         pipeline_mode=pl.Buffered(buffer_count=1)),
        ],
        out_specs=pl.BlockSpec((rows_f, 128), lambda f: (f, 0),
                               pipeline_mode=pl.Buffered(buffer_count=1)),
        out_shape=jax.ShapeDtypeStruct((f_all * v // pack, 128), jnp.float32),
        compiler_params=pltpu.CompilerParams(
            vmem_limit_bytes=100 * 1024 * 1024),
    )(tt, s_full, s_tail)


def _sc_gather(tflat, idx_flat):
    n_idx = idx_flat.shape[1]
    d = tflat.shape[1]
    mesh = plsc.VectorSubcoreMesh(
        core_axis_name="core", subcore_axis_name="subcore"
    )

    @functools.partial(
        pl.kernel,
        out_type=jax.ShapeDtypeStruct((n_idx, d), tflat.dtype),
        mesh=mesh,
        compiler_params=pltpu.CompilerParams(use_tc_tiling_on_sc=False),
    )
    def _kernel(x_hbm, i_hbm, o_hbm):
        def body(i_vmem, o_vmem):
            pltpu.sync_copy(x_hbm.at[i_vmem.at[0]], o_vmem)

        pltpu.emit_pipeline(
            body,
            grid=(n_idx // _GATHER_WINDOW,),
            in_specs=[
                pl.BlockSpec((1, _GATHER_WINDOW), index_map=lambda i: (0, i))
            ],
            out_specs=[
                pl.BlockSpec((_GATHER_WINDOW, d), index_map=lambda i: (i, 0))
            ],
            core_axis_name=("core", "subcore"),
            dimension_semantics=(pltpu.PARALLEL,),
        )(i_hbm, o_hbm)

    return _kernel(tflat, idx_flat)


def _finish_body(x_ref, hb_ref, d_ref, t00_ref, a1_ref, a2_ref,
                 gp_ref, gf_ref, gd_ref, bp_ref, bf_ref, bd_ref, o_ref,
                 *, hist, n_fields, d_emb, ln_dim):
    nf_cols = n_fields * d_emb
    x = x_ref[...]
    xf = x[:, :nf_cols]
    sums = jnp.dot(x, a1_ref[...], preferred_element_type=jnp.float32)
    h1u = sums[:, :d_emb]
    sf = sums[:, d_emb:]
    qf = jnp.dot(xf * xf, a2_ref[...], preferred_element_type=jnp.float32)

    hb = hb_ref[...]
    cnt = jnp.sum((hb > 0).astype(jnp.float32), axis=1, keepdims=True)
    n0 = hist - cnt
    h1 = h1u - n0 * t00_ref[...]
    pooled = h1 / (cnt + 1e-8)

    s = pooled + sf
    q = pooled * pooled + qf
    dn = d_ref[...]
    sum_x = jnp.sum(s, axis=1, keepdims=True) + jnp.sum(
        dn, axis=1, keepdims=True)
    sumsq_x = jnp.sum(q, axis=1, keepdims=True) + jnp.sum(
        dn * dn, axis=1, keepdims=True)
    mu = sum_x / ln_dim
    var = sumsq_x / ln_dim - mu * mu
    rstd = jax.lax.rsqrt(var + 1e-5)
    cross = 0.5 * (jnp.sum(s * s, axis=1, keepdims=True)
                   - jnp.sum(q, axis=1, keepdims=True))

    ln_p = gp_ref[...] * ((pooled - mu) * rstd) + bp_ref[...]
    ln_f = gf_ref[...] * ((xf - mu) * rstd) + bf_ref[...]
    ln_d = gd_ref[...] * ((dn - mu) * rstd) + bd_ref[...]
    o_ref[...] = jnp.concatenate([ln_p, ln_f, ln_d, cross], axis=1)


def kernel(tables, gamma, beta, dense, ids, hist_ids):
    f_all, v, d = tables.shape
    b = dense.shape[0]
    n_fields = f_all - 1
    hist = hist_ids.shape[1]
    n_dense = dense.shape[1]
    row = (n_fields + hist) * d
    nf_cols = n_fields * d
    ln_dim = f_all * d + n_dense

    ids32 = ids.astype(jnp.int32)
    hist32 = hist_ids.astype(jnp.int32)
    offs = (jnp.arange(1, f_all, dtype=jnp.int32) * v)[None, :]
    idx_all = jnp.concatenate([ids32 + offs, hist32], axis=1)
    t128 = _transpose_tables(tables)
    tflat = t128.reshape(f_all * v, d)

    eye = jnp.eye(d, dtype=jnp.float32)
    zed = jnp.zeros((d, d), dtype=jnp.float32)
    a1 = jnp.concatenate(
        [
            jnp.tile(jnp.concatenate([zed, eye], axis=1), (n_fields, 1)),
            jnp.tile(jnp.concatenate([eye, zed], axis=1), (hist, 1)),
        ],
        axis=0,
    )
    a2 = jnp.tile(eye, (n_fields, 1))
    t00 = tables[0, 0].reshape(1, d)
    gp, gf, gd = (gamma[:d].reshape(1, d),
                  gamma[d:d + nf_cols].reshape(1, nf_cols),
                  gamma[d + nf_cols:].reshape(1, n_dense))
    bp, bf, bd = (beta[:d].reshape(1, d),
                  beta[d:d + nf_cols].reshape(1, nf_cols),
                  beta[d + nf_cols:].reshape(1, n_dense))

    body = functools.partial(_finish_body, hist=float(hist),
                             n_fields=n_fields, d_emb=d, ln_dim=float(ln_dim))
    const = lambda shape: pl.BlockSpec(shape, lambda i: (0, 0))

    bc = b // _CHUNKS
    outs = []
    for c in range(_CHUNKS):
        sl = slice(c * bc, (c + 1) * bc)
        idx_flat = idx_all[sl].reshape(1, bc * (n_fields + hist))
        gathered = _sc_gather(tflat, idx_flat)
        x = gathered.reshape(bc, row)
        out_c = pl.pallas_call(
            body,
            grid=(bc // _BT,),
            in_specs=[
                pl.BlockSpec((_BT, row), lambda i: (i, 0)),
                pl.BlockSpec((_BT, hist), lambda i: (i, 0)),
                pl.BlockSpec((_BT, n_dense), lambda i: (i, 0)),
                const((1, d)),
                const((row, 2 * d)),
                const((nf_cols, d)),
                const((1, d)),
                const((1, nf_cols)),
                const((1, n_dense)),
                const((1, d)),
                const((1, nf_cols)),
                const((1, n_dense)),
            ],
            out_specs=pl.BlockSpec((_BT, ln_dim + 1), lambda i: (i, 0)),
            out_shape=jax.ShapeDtypeStruct((bc, ln_dim + 1), jnp.float32),
        )(x, hist32[sl], dense[sl], t00, a1, a2, gp, gf, gd, bp, bf, bd)
        outs.append(out_c)
    return jnp.concatenate(outs, axis=0)
